# baseline (speedup 1.0000x reference)
import functools

import jax
import jax.numpy as jnp
from jax import lax
from jax.experimental import pallas as pl
from jax.experimental.pallas import tpu as pltpu
from jax.experimental.pallas import tpu_sc as plsc

ORIG_VOCAB = 1000000
NEW_VOCAB = 1000
D = 64
L = 16
NC, NS = 2, 16
NW = NC * NS
HIST = 200
BB = 128


def _sc_body(ids_hbm, orig_hbm, new_hbm, out_hbm, newtbl_v, idxT_v, stage_v,
             cid_v, rows_v, trT_v, sem):
    wid = lax.axis_index("s") * NC + lax.axis_index("c")
    batch = ids_hbm.shape[0]
    blocks_per_w = batch // BB // NW
    iota = lax.iota(jnp.int32, L)

    pltpu.sync_copy(new_hbm, newtbl_v)

    def blk_body(blk, carry):
        bt = wid * blocks_per_w + blk
        b0 = bt * BB

        def stage_body(s, c):
            pltpu.sync_copy(ids_hbm.at[pl.ds(b0 + s * L, L)], stage_v)

            def t_body(t, c2):
                vals = plsc.load_gather(stage_v, [iota, jnp.full((L,), t, jnp.int32)])
                idxT_v[t, pl.ds(s * L, L)] = vals
                return c2

            lax.fori_loop(0, HIST, t_body, 0)
            return c

        lax.fori_loop(0, BB // L, stage_body, 0)

        def t_body2(t, c):
            for g in range(BB // L):
                v = idxT_v[t, pl.ds(g * L, L)]
                cid_v[pl.ds(g * L, L)] = jnp.minimum(v, ORIG_VOCAB - 1)
            pltpu.async_copy(orig_hbm.at[cid_v], rows_v, sem).wait()

            for d in range(D):
                dt, ds = d // 8, d % 8
                dvec = jnp.full((L,), d, jnp.int32)
                for g in range(BB // L):
                    vals = plsc.load_gather(rows_v, [iota + g * L, dvec])
                    trT_v[dt, ds, pl.ds(g * L, L)] = vals

            for g in range(BB // L):
                v = idxT_v[t, pl.ds(g * L, L)]
                m = v >= ORIG_VOCAB
                cnt = plsc.all_reduce_population_count(m)[0]

                @pl.when(cnt > 0)
                def _():
                    nid = jnp.where(m, v - ORIG_VOCAB, 0)
                    bvec = iota + g * L
                    for d in range(D):
                        vals = plsc.load_gather(
                            newtbl_v, [nid, jnp.full((L,), d, jnp.int32)])
                        plsc.store_scatter(
                            trT_v,
                            [jnp.full((L,), d // 8, jnp.int32),
                             jnp.full((L,), d % 8, jnp.int32), bvec],
                            vals, mask=m)

            pltpu.sync_copy(trT_v, out_hbm.at[t, :, bt])
            return c

        lax.fori_loop(0, HIST, t_body2, 0)
        return carry

    lax.fori_loop(0, blocks_per_w, blk_body, 0)


@functools.lru_cache(maxsize=None)
def _make_sc_call(batch, hist):
    mesh = plsc.VectorSubcoreMesh(core_axis_name="c", subcore_axis_name="s")
    return pl.kernel(
        _sc_body,
        out_type=jax.ShapeDtypeStruct((hist, D // 8, batch // BB, 8, BB),
                                      jnp.float32),
        mesh=mesh,
        scratch_types=[
            pltpu.VMEM((NEW_VOCAB, D), jnp.float32),
            pltpu.VMEM((HIST, BB), jnp.int32),
            pltpu.VMEM((L, HIST), jnp.int32),
            pltpu.VMEM((BB,), jnp.int32),
            pltpu.VMEM((BB, D), jnp.float32),
            pltpu.VMEM((D // 8, 8, BB), jnp.float32),
            pltpu.SemaphoreType.DMA,
        ],
        compiler_params=pltpu.CompilerParams(
            use_tc_tiling_on_sc=False, needs_layout_passes=False),
    )


@jax.jit
def kernel(input_ids, orig_table, new_table):
    b, h = input_ids.shape
    ids = input_ids.astype(jnp.int32)
    out4 = _make_sc_call(b, h)(ids, orig_table, new_table)
    x = lax.transpose(out4, (2, 4, 0, 1, 3))
    return x.reshape(b, h, D)


# speedup vs baseline: 1.1062x; 1.1062x over previous
import functools

import jax
import jax.numpy as jnp
from jax import lax
from jax.experimental import pallas as pl
from jax.experimental.pallas import tpu as pltpu
from jax.experimental.pallas import tpu_sc as plsc

D = 64
L = 16
NC, NS = 2, 16
NW = NC * NS
HIST = 200
BB = 128
NG = BB // L


def _sc_body(ids_hbm, tbl_hbm, out_hbm, idxT_v, stage_v, rows2, trT2,
             gsem, wsem):
    wid = lax.axis_index("s") * NC + lax.axis_index("c")
    batch = ids_hbm.shape[0]
    blocks_per_w = batch // BB // NW
    iota = lax.iota(jnp.int32, L)
    bvecs = [iota + g * L for g in range(NG)]

    def _transpose(s):
        for d in range(D):
            dvec = jnp.full((L,), d, jnp.int32)
            for g in range(NG):
                vals = plsc.load_gather(rows2.at[s], [bvecs[g], dvec])
                trT2[s, d // 8, d % 8, pl.ds(g * L, L)] = vals

    def blk_body(blk, carry):
        bt = wid * blocks_per_w + blk
        b0 = bt * BB

        def stage_body(st, c):
            pltpu.sync_copy(ids_hbm.at[pl.ds(b0 + st * L, L)], stage_v)

            def t_body(t, c2):
                vals = plsc.load_gather(
                    stage_v, [iota, jnp.full((L,), t, jnp.int32)])
                idxT_v[t, pl.ds(st * L, L)] = vals
                return c2

            lax.fori_loop(0, HIST, t_body, 0)
            return c

        lax.fori_loop(0, NG, stage_body, 0)

        def _step(t, s):
            @pl.when(t < HIST - 1)
            def _():
                pltpu.async_copy(
                    tbl_hbm.at[idxT_v.at[t + 1]], rows2.at[1 - s], gsem)

            pltpu.make_async_copy(
                tbl_hbm.at[idxT_v.at[0]], rows2.at[s], gsem).wait()

            @pl.when(t >= 2)
            def _():
                pltpu.make_async_copy(
                    trT2.at[s], out_hbm.at[0, :, bt], wsem).wait()

            _transpose(s)
            pltpu.async_copy(trT2.at[s], out_hbm.at[t, :, bt], wsem)

        pltpu.async_copy(tbl_hbm.at[idxT_v.at[0]], rows2.at[0], gsem)

        def pair_body(i, c):
            _step(2 * i, 0)
            _step(2 * i + 1, 1)
            return c

        lax.fori_loop(0, HIST // 2, pair_body, 0)

        for s in range(2):
            pltpu.make_async_copy(
                trT2.at[s], out_hbm.at[0, :, bt], wsem).wait()
        return carry

    lax.fori_loop(0, blocks_per_w, blk_body, 0)


@functools.lru_cache(maxsize=None)
def _make_sc_call(batch, hist):
    mesh = plsc.VectorSubcoreMesh(core_axis_name="c", subcore_axis_name="s")
    return pl.kernel(
        _sc_body,
        out_type=jax.ShapeDtypeStruct((hist, D // 8, batch // BB, 8, BB),
                                      jnp.float32),
        mesh=mesh,
        scratch_types=[
            pltpu.VMEM((HIST, BB), jnp.int32),
            pltpu.VMEM((L, HIST), jnp.int32),
            pltpu.VMEM((2, BB, D), jnp.float32),
            pltpu.VMEM((2, D // 8, 8, BB), jnp.float32),
            pltpu.SemaphoreType.DMA,
            pltpu.SemaphoreType.DMA,
        ],
        compiler_params=pltpu.CompilerParams(
            use_tc_tiling_on_sc=False, needs_layout_passes=False),
    )


@jax.jit
def kernel(input_ids, orig_table, new_table):
    b, h = input_ids.shape
    ids = input_ids.astype(jnp.int32)
    table = jnp.concatenate([orig_table, new_table], axis=0)
    out4 = _make_sc_call(b, h)(ids, table)
    x = lax.transpose(out4, (2, 4, 0, 1, 3))
    return x.reshape(b, h, D)


# speedup vs baseline: 1.9455x; 1.7587x over previous
import functools

import jax
import jax.numpy as jnp
from jax import lax
from jax.experimental import pallas as pl
from jax.experimental.pallas import tpu as pltpu
from jax.experimental.pallas import tpu_sc as plsc

D = 64
L = 16
NC, NS = 2, 16
NW = NC * NS
HIST = 200
BB = 128
NG = BB // L


def _sc_body(ids_hbm, tbl_hbm, out_hbm, idxT_v, stage_v, rows2, trT2,
             gsem, wsem):
    wid = lax.axis_index("s") * NC + lax.axis_index("c")
    batch = ids_hbm.shape[0]
    blocks_per_w = batch // BB // NW
    iota = lax.iota(jnp.int32, L)
    bvecs = [iota + g * L for g in range(NG)]

    def _transpose(s):
        @plsc.parallel_loop(0, D, unroll=8)
        def _(d):
            dvec = jnp.full((L,), d, jnp.int32)
            dt = d // 8
            dsub = d % 8
            for g in range(NG):
                vals = plsc.load_gather(rows2.at[s], [bvecs[g], dvec])
                trT2[s, dt, dsub, pl.ds(g * L, L)] = vals

    def blk_body(blk, carry):
        bt = wid * blocks_per_w + blk
        b0 = bt * BB

        def stage_body(st, c):
            pltpu.sync_copy(ids_hbm.at[pl.ds(b0 + st * L, L)], stage_v)

            @plsc.parallel_loop(0, HIST, unroll=8)
            def _(t):
                vals = plsc.load_gather(
                    stage_v, [iota, jnp.full((L,), t, jnp.int32)])
                idxT_v[t, pl.ds(st * L, L)] = vals

            return c

        lax.fori_loop(0, NG, stage_body, 0)

        def _step(t, s):
            @pl.when(t < HIST - 1)
            def _():
                pltpu.async_copy(
                    tbl_hbm.at[idxT_v.at[t + 1]], rows2.at[1 - s], gsem)

            pltpu.make_async_copy(
                tbl_hbm.at[idxT_v.at[0]], rows2.at[s], gsem).wait()

            @pl.when(t >= 2)
            def _():
                pltpu.make_async_copy(
                    trT2.at[s], out_hbm.at[0, :, bt], wsem).wait()

            _transpose(s)
            pltpu.async_copy(trT2.at[s], out_hbm.at[t, :, bt], wsem)

        pltpu.async_copy(tbl_hbm.at[idxT_v.at[0]], rows2.at[0], gsem)

        def pair_body(i, c):
            _step(2 * i, 0)
            _step(2 * i + 1, 1)
            return c

        lax.fori_loop(0, HIST // 2, pair_body, 0)

        for s in range(2):
            pltpu.make_async_copy(
                trT2.at[s], out_hbm.at[0, :, bt], wsem).wait()
        return carry

    lax.fori_loop(0, blocks_per_w, blk_body, 0)


@functools.lru_cache(maxsize=None)
def _make_sc_call(batch, hist):
    mesh = plsc.VectorSubcoreMesh(core_axis_name="c", subcore_axis_name="s")
    return pl.kernel(
        _sc_body,
        out_type=jax.ShapeDtypeStruct((hist, D // 8, batch // BB, 8, BB),
                                      jnp.float32),
        mesh=mesh,
        scratch_types=[
            pltpu.VMEM((HIST, BB), jnp.int32),
            pltpu.VMEM((L, HIST), jnp.int32),
            pltpu.VMEM((2, BB, D), jnp.float32),
            pltpu.VMEM((2, D // 8, 8, BB), jnp.float32),
            pltpu.SemaphoreType.DMA,
            pltpu.SemaphoreType.DMA,
        ],
        compiler_params=pltpu.CompilerParams(
            use_tc_tiling_on_sc=False, needs_layout_passes=False),
    )


@jax.jit
def kernel(input_ids, orig_table, new_table):
    b, h = input_ids.shape
    ids = input_ids.astype(jnp.int32)
    table = jnp.concatenate([orig_table, new_table], axis=0)
    out4 = _make_sc_call(b, h)(ids, table)
    x = lax.transpose(out4, (2, 4, 0, 1, 3))
    return x.reshape(b, h, D)


# speedup vs baseline: 2.0288x; 1.0428x over previous
import functools

import jax
import jax.numpy as jnp
from jax import lax
from jax.experimental import pallas as pl
from jax.experimental.pallas import tpu as pltpu
from jax.experimental.pallas import tpu_sc as plsc

D = 64
L = 16
NC, NS = 2, 16
NW = NC * NS
HIST = 200
BB = 128
NG = BB // L


def _sc_body(ids_hbm, tbl_hbm, out_hbm, idxT_v, stage_v, rows2, trT2,
             gsem, wsem):
    wid = lax.axis_index("s") * NC + lax.axis_index("c")
    batch = ids_hbm.shape[0]
    blocks_per_w = batch // BB // NW
    iota = lax.iota(jnp.int32, L)
    bvecs = [iota + g * L for g in range(NG)]

    def _transpose(s):
        for g in range(NG):
            @plsc.parallel_loop(0, D, unroll=8)
            def _(d, _g=g):
                dvec = jnp.full((L,), d, jnp.int32)
                vals = plsc.load_gather(rows2.at[s], [bvecs[_g], dvec])
                trT2[s, d // 8, d % 8, pl.ds(_g * L, L)] = vals

    def blk_body(blk, carry):
        bt = wid * blocks_per_w + blk
        b0 = bt * BB

        def stage_body(st, c):
            pltpu.sync_copy(ids_hbm.at[pl.ds(b0 + st * L, L)], stage_v)

            @plsc.parallel_loop(0, HIST, unroll=8)
            def _(t):
                vals = plsc.load_gather(
                    stage_v, [iota, jnp.full((L,), t, jnp.int32)])
                idxT_v[t, pl.ds(st * L, L)] = vals

            return c

        lax.fori_loop(0, NG, stage_body, 0)

        def _step(t, s):
            @pl.when(t < HIST - 1)
            def _():
                pltpu.async_copy(
                    tbl_hbm.at[idxT_v.at[t + 1]], rows2.at[1 - s], gsem)

            pltpu.make_async_copy(
                tbl_hbm.at[idxT_v.at[0]], rows2.at[s], gsem).wait()

            @pl.when(t >= 2)
            def _():
                pltpu.make_async_copy(
                    trT2.at[s], out_hbm.at[0, :, bt], wsem).wait()

            _transpose(s)
            pltpu.async_copy(trT2.at[s], out_hbm.at[t, :, bt], wsem)

        pltpu.async_copy(tbl_hbm.at[idxT_v.at[0]], rows2.at[0], gsem)

        def pair_body(i, c):
            _step(2 * i, 0)
            _step(2 * i + 1, 1)
            return c

        lax.fori_loop(0, HIST // 2, pair_body, 0)

        for s in range(2):
            pltpu.make_async_copy(
                trT2.at[s], out_hbm.at[0, :, bt], wsem).wait()
        return carry

    lax.fori_loop(0, blocks_per_w, blk_body, 0)


@functools.lru_cache(maxsize=None)
def _make_sc_call(batch, hist):
    mesh = plsc.VectorSubcoreMesh(core_axis_name="c", subcore_axis_name="s")
    return pl.kernel(
        _sc_body,
        out_type=jax.ShapeDtypeStruct((hist, D // 8, batch // BB, 8, BB),
                                      jnp.float32),
        mesh=mesh,
        scratch_types=[
            pltpu.VMEM((HIST, BB), jnp.int32),
            pltpu.VMEM((L, HIST), jnp.int32),
            pltpu.VMEM((2, BB, D), jnp.float32),
            pltpu.VMEM((2, D // 8, 8, BB), jnp.float32),
            pltpu.SemaphoreType.DMA,
            pltpu.SemaphoreType.DMA,
        ],
        compiler_params=pltpu.CompilerParams(
            use_tc_tiling_on_sc=False, needs_layout_passes=False),
    )


@jax.jit
def kernel(input_ids, orig_table, new_table):
    b, h = input_ids.shape
    ids = input_ids.astype(jnp.int32)
    table = jnp.concatenate([orig_table, new_table], axis=0)
    out4 = _make_sc_call(b, h)(ids, table)
    x = lax.transpose(out4, (2, 4, 0, 1, 3))
    return x.reshape(b, h, D)


# speedup vs baseline: 4.3474x; 2.1428x over previous
import functools

import jax
import jax.numpy as jnp
from jax import lax
from jax.experimental import pallas as pl
from jax.experimental.pallas import tpu as pltpu
from jax.experimental.pallas import tpu_sc as plsc

D = 64
L = 16
NC, NS = 2, 16
NW = NC * NS
HIST = 200
BB = 128
NG = BB // L


def _sc_body(ids_hbm, tbl_hbm, out_hbm, idxT_v, stage_v, rows2, trT2,
             gsem, wsem):
    wid = lax.axis_index("s") * NC + lax.axis_index("c")
    batch = ids_hbm.shape[0]
    blocks_per_w = batch // BB // NW
    iota = lax.iota(jnp.int32, L)
    bvecs = [iota + g * L for g in range(NG)]

    dchunk = [((iota + k * L) // 8, (iota + k * L) % 8) for k in range(D // L)]

    def _transpose(s):
        @plsc.parallel_loop(0, BB, unroll=8)
        def _(b):
            bvec = jnp.full((L,), b, jnp.int32)
            for k in range(D // L):
                vals = rows2[s, b, pl.ds(k * L, L)]
                plsc.store_scatter(
                    trT2.at[s], [dchunk[k][0], dchunk[k][1], bvec], vals)

    def blk_body(blk, carry):
        bt = wid * blocks_per_w + blk
        b0 = bt * BB

        def stage_body(st, c):
            pltpu.sync_copy(ids_hbm.at[pl.ds(b0 + st * L, L)], stage_v)

            @plsc.parallel_loop(0, HIST, unroll=8)
            def _(t):
                vals = plsc.load_gather(
                    stage_v, [iota, jnp.full((L,), t, jnp.int32)])
                idxT_v[t, pl.ds(st * L, L)] = vals

            return c

        lax.fori_loop(0, NG, stage_body, 0)

        def _step(t, s):
            @pl.when(t < HIST - 1)
            def _():
                pltpu.async_copy(
                    tbl_hbm.at[idxT_v.at[t + 1]], rows2.at[1 - s], gsem)

            pltpu.make_async_copy(
                tbl_hbm.at[idxT_v.at[0]], rows2.at[s], gsem).wait()

            @pl.when(t >= 2)
            def _():
                pltpu.make_async_copy(
                    trT2.at[s, :, :, pl.ds(0, BB)],
                    out_hbm.at[0, :, bt], wsem).wait()

            _transpose(s)
            pltpu.async_copy(
                trT2.at[s, :, :, pl.ds(0, BB)], out_hbm.at[t, :, bt], wsem)

        pltpu.async_copy(tbl_hbm.at[idxT_v.at[0]], rows2.at[0], gsem)

        def pair_body(i, c):
            _step(2 * i, 0)
            _step(2 * i + 1, 1)
            return c

        lax.fori_loop(0, HIST // 2, pair_body, 0)

        for s in range(2):
            pltpu.make_async_copy(
                trT2.at[s, :, :, pl.ds(0, BB)],
                out_hbm.at[0, :, bt], wsem).wait()
        return carry

    lax.fori_loop(0, blocks_per_w, blk_body, 0)


@functools.lru_cache(maxsize=None)
def _make_sc_call(batch, hist):
    mesh = plsc.VectorSubcoreMesh(core_axis_name="c", subcore_axis_name="s")
    return pl.kernel(
        _sc_body,
        out_type=jax.ShapeDtypeStruct((hist, D // 8, batch // BB, 8, BB),
                                      jnp.float32),
        mesh=mesh,
        scratch_types=[
            pltpu.VMEM((HIST, BB), jnp.int32),
            pltpu.VMEM((L, HIST), jnp.int32),
            pltpu.VMEM((2, BB, D), jnp.float32),
            pltpu.VMEM((2, D // 8, 8, BB + 1), jnp.float32),
            pltpu.SemaphoreType.DMA,
            pltpu.SemaphoreType.DMA,
        ],
        compiler_params=pltpu.CompilerParams(
            use_tc_tiling_on_sc=False, needs_layout_passes=False),
    )


@jax.jit
def kernel(input_ids, orig_table, new_table):
    b, h = input_ids.shape
    ids = input_ids.astype(jnp.int32)
    table = jnp.concatenate([orig_table, new_table], axis=0)
    out4 = _make_sc_call(b, h)(ids, table)
    x = lax.transpose(out4, (2, 4, 0, 1, 3))
    return x.reshape(b, h, D)
